# Initial kernel scaffold; baseline (speedup 1.0000x reference)
import jax
import jax.numpy as jnp
from jax.experimental import pallas as pl


def kernel(x, adj, W1, b1, W2, b2):
    raise NotImplementedError("write your pallas kernel here")


# speedup vs baseline: 1.0249x; 1.0249x over previous
import functools

import jax
import jax.numpy as jnp
from jax.experimental import pallas as pl


def _layer_kernel(adj_ref, m_ref, w_ref, b_ref, out_ref, *, relu):
    g = jnp.dot(adj_ref[...], m_ref[...], preferred_element_type=jnp.float32)
    h = jnp.dot(g, w_ref[...], preferred_element_type=jnp.float32) + b_ref[...]
    if relu:
        h = jnp.maximum(h, 0.0)
    out_ref[...] = h


def _layer(adj, m, w, b, relu, bm):
    n = adj.shape[0]
    d = m.shape[1]
    return pl.pallas_call(
        functools.partial(_layer_kernel, relu=relu),
        grid=(n // bm,),
        in_specs=[
            pl.BlockSpec((bm, n), lambda i: (i, 0)),
            pl.BlockSpec((n, d), lambda i: (0, 0)),
            pl.BlockSpec((d, d), lambda i: (0, 0)),
            pl.BlockSpec((1, d), lambda i: (0, 0)),
        ],
        out_specs=pl.BlockSpec((bm, d), lambda i: (i, 0)),
        out_shape=jax.ShapeDtypeStruct((n, d), jnp.float32),
    )(adj, m, w, b)


def kernel(x, adj, W1, b1, W2, b2):
    b1r = b1.reshape(1, -1)
    b2r = b2.reshape(1, -1)
    h = _layer(adj, x, W1, b1r, relu=True, bm=400)
    return _layer(adj, h, W2, b2r, relu=False, bm=400)
